# baseline (speedup 1.0000x reference)
import functools

import jax
import jax.numpy as jnp
from jax import lax
from jax.experimental import pallas as pl
from jax.experimental.pallas import tpu as pltpu
from jax.experimental.pallas import tpu_sc as plsc

_NA = 3
_CS = 4
_NB = 32
_CH = _NA * 6
_N = 26 * 26 + 52 * 52 + 104 * 104
_FWS = (26, 52, 104)
_OFFS = (0, 676, 3380)
_ANCH = (
    ((116.0, 90.0), (156.0, 198.0), (373.0, 326.0)),
    ((30.0, 61.0), (62.0, 45.0), (59.0, 119.0)),
    ((10.0, 13.0), (16.0, 30.0), (33.0, 23.0)),
)
_EPS = 1e-7
_NOOBJ = 100.0
_IGN = 0.5
_NSLOT = 11
_NAUX = 12
_TOT = 96 * 50 * _NSLOT
_NW = 32
_KCH = 13
_NP = 14336


def _idx_body(tb_ref, idx_ref, aux_ref):
    bx = tb_ref[1]
    by = tb_ref[2]
    bw = tb_ref[3]
    bh = tb_ref[4]
    valid = bw > 0.0
    gw = bw * 832.0
    gh = bh * 832.0
    row_b = lax.broadcasted_iota(jnp.int32, (_NB, 50), 0)
    bq = (row_b // _CS) * _CS
    ri50 = lax.broadcasted_iota(jnp.int32, (50, 50), 0)
    ci50 = lax.broadcasted_iota(jnp.int32, (50, 50), 1)
    lower = (ci50 < ri50)[None]
    f32 = lambda v: v.astype(jnp.float32)
    for s in range(3):
        fw = _FWS[s]
        ns = fw * fw
        iou = []
        for a in range(3):
            aw, ah = _ANCH[s][a]
            inter = jnp.minimum(gw, aw) * jnp.minimum(gh, ah)
            union = gw * gh + aw * ah - inter
            iou.append(inter / (union + 1e-16))
        best = jnp.where(
            iou[0] >= iou[1],
            jnp.where(iou[0] >= iou[2], 0, 2),
            jnp.where(iou[1] >= iou[2], 1, 2),
        ).astype(jnp.int32)
        gx = bx * float(fw)
        gy = by * float(fw)
        fgx = jnp.floor(gx)
        fgy = jnp.floor(gy)
        gi = jnp.clip(fgx, 0.0, float(fw - 1)).astype(jnp.int32)
        gj = jnp.clip(fgy, 0.0, float(fw - 1)).astype(jnp.int32)
        loc = gj * fw + gi
        gloc = _OFFS[s] + loc
        keyo = best * ns + loc
        awb = jnp.where(best == 0, _ANCH[s][0][0],
                        jnp.where(best == 1, _ANCH[s][1][0], _ANCH[s][2][0]))
        ahb = jnp.where(best == 0, _ANCH[s][0][1],
                        jnp.where(best == 1, _ANCH[s][1][1], _ANCH[s][2][1]))
        same = keyo[:, :, None] == keyo[:, None, :]
        prior = same & valid[:, None, :] & lower
        uo = valid & jnp.logical_not(jnp.any(prior, axis=2))
        ufl = []
        for a in range(3):
            ign_a = (iou[a] > _IGN) & valid
            obj_a = (best == a) & valid
            ufl.append(ign_a | obj_a)
        same_loc = loc[:, :, None] == loc[:, None, :]
        fu = []
        for a in range(3):
            p_a = same_loc & ufl[a][:, None, :] & lower
            fu.append(ufl[a] & jnp.logical_not(jnp.any(p_a, axis=2)))
        for c in range(4):
            idx_ref[s, c] = ((best * 6 + c) * _NB + row_b) * _NP + gloc
        for a in range(3):
            idx_ref[s, 4 + a] = ((a * 6 + 4) * _NB + row_b) * _NP + gloc
        for j in range(4):
            idx_ref[s, 7 + j] = ((best * 6 + 5) * _NB + bq + j) * _NP + gloc
        aux_ref[s, 0] = gx - fgx
        aux_ref[s, 1] = gy - fgy
        aux_ref[s, 2] = jnp.log(jnp.maximum(gw, 1e-6) / awb)
        aux_ref[s, 3] = jnp.log(jnp.maximum(gh, 1e-6) / ahb)
        aux_ref[s, 4] = f32(uo)
        aux_ref[s, 5] = f32(valid)
        for a in range(3):
            aux_ref[s, 6 + a] = f32(fu[a])
            aux_ref[s, 9 + a] = f32(best == a)


def _build_idx(tbt):
    return pl.pallas_call(
        _idx_body,
        in_specs=[pl.BlockSpec((5, _NB, 50), lambda: (0, 0, 0))],
        out_specs=[
            pl.BlockSpec((3, _NSLOT, _NB, 50), lambda: (0, 0, 0, 0)),
            pl.BlockSpec((3, _NAUX, _NB, 50), lambda: (0, 0, 0, 0)),
        ],
        out_shape=[
            jax.ShapeDtypeStruct((3, _NSLOT, _NB, 50), jnp.int32),
            jax.ShapeDtypeStruct((3, _NAUX, _NB, 50), jnp.float32),
        ],
    )(tbt)


@functools.cache
def _sc_gather():
    mesh = plsc.VectorSubcoreMesh(
        core_axis_name="c", subcore_axis_name="s", num_cores=2, num_subcores=16)

    @functools.partial(
        pl.kernel,
        out_type=jax.ShapeDtypeStruct((_NW, _KCH, 128), jnp.float32),
        mesh=mesh,
        scratch_types=[
            pltpu.VMEM((_KCH, 128), jnp.int32),
            pltpu.VMEM((_KCH, 128), jnp.float32),
            pltpu.SemaphoreType.DMA,
        ],
    )
    def gather(flat_hbm, idx_hbm, out_hbm, idx_v, val_v, sem):
        wid = lax.axis_index("s") * 2 + lax.axis_index("c")
        pltpu.sync_copy(idx_hbm.at[wid], idx_v)
        copies = [
            pltpu.make_async_copy(flat_hbm.at[idx_v.at[j]], val_v.at[j], sem)
            for j in range(_KCH)
        ]
        for cp in copies:
            cp.start()
        for cp in copies:
            cp.wait()
        pltpu.sync_copy(val_v, out_hbm.at[wid])

    return gather


def _lin_body(z_ref, out_ref):
    pad = jnp.full((_NP - _N,), -100.0, jnp.float32)
    for q in range(2):
        z = z_ref[q]
        for b in range(_NB):
            out_ref[pl.ds((q * _NB + b) * _NP, _N)] = z[b]
            out_ref[pl.ds((q * _NB + b) * _NP + _N, _NP - _N)] = pad


def _linearize(out_t):
    return pl.pallas_call(
        _lin_body,
        grid=(_CH // 2,),
        in_specs=[pl.BlockSpec((2, _NB, _N), lambda g: (g, 0, 0))],
        out_specs=pl.BlockSpec((2 * _NB * _NP,), lambda g: (g,)),
        out_shape=jax.ShapeDtypeStruct((_CH * _NB * _NP,), jnp.float32),
    )(out_t)


def _dense_body(z0_ref, z1_ref, z2_ref, out_ref):
    g = pl.program_id(0)
    v = jnp.float32(0.0)
    for r in (z0_ref, z1_ref, z2_ref):
        v += jnp.sum(jnp.minimum(jnp.log1p(jnp.exp(r[...])), 15.942385))

    @pl.when(g == 0)
    def _():
        out_ref[0, 0] = 0.0

    out_ref[0, 0] += _NOOBJ * v


def _dense_sum(flat):
    return pl.pallas_call(
        _dense_body,
        grid=(_NB,),
        in_specs=[
            pl.BlockSpec((_NP,), lambda g: (4 * _NB + g,)),
            pl.BlockSpec((_NP,), lambda g: (10 * _NB + g,)),
            pl.BlockSpec((_NP,), lambda g: (16 * _NB + g,)),
        ],
        out_specs=pl.BlockSpec(memory_space=pltpu.SMEM),
        out_shape=jax.ShapeDtypeStruct((1, 1), jnp.float32),
    )(flat, flat, flat)


def _combine_body(gat_ref, aux_ref, dense_ref, out_ref):
    zx = gat_ref[0]
    zy = gat_ref[1]
    zw = gat_ref[2]
    zh = gat_ref[3]
    zc = [gat_ref[4 + a] for a in range(3)]
    cl = [gat_ref[7 + j] for j in range(4)]
    tx = aux_ref[0]
    ty = aux_ref[1]
    tw = aux_ref[2]
    th = aux_ref[3]
    uo = aux_ref[4]
    valid = aux_ref[5]
    fu = [aux_ref[6 + a] for a in range(3)]
    oh = [aux_ref[9 + a] for a in range(3)]
    zca = oh[0] * zc[0] + oh[1] * zc[1] + oh[2] * zc[2]
    pa = jnp.clip(jax.nn.sigmoid(zca), _EPS, 1.0 - _EPS)
    obj_term = uo * (
        -jnp.log(pa)
        + (jax.nn.sigmoid(zx) - tx) ** 2
        + (jax.nn.sigmoid(zy) - ty) ** 2
        + (zw - tw) ** 2
        + (zh - th) ** 2
    )
    ucorr = jnp.zeros_like(zx)
    for a in range(3):
        pca = jnp.clip(jax.nn.sigmoid(zc[a]), _EPS, 1.0 - _EPS)
        ucorr = ucorr + fu[a] * (-jnp.log(1.0 - pca))
    row = lax.broadcasted_iota(jnp.int32, (96, 50), 0)
    tgt = row % _CS
    m = jnp.maximum(jnp.maximum(cl[0], cl[1]), jnp.maximum(cl[2], cl[3]))
    se = jnp.zeros_like(m)
    ctgt = jnp.zeros_like(m)
    for j in range(4):
        se = se + jnp.exp(cl[j] - m)
        ctgt = ctgt + jnp.where(tgt == j, cl[j], 0.0)
    lse = m + jnp.log(se)
    cls_term = valid * (lse - ctgt)
    total = jnp.sum(obj_term - _NOOBJ * ucorr + cls_term)
    out_ref[0, 0] = dense_ref[0, 0] + total


def _combine(gat_t, aux_t, dense):
    return pl.pallas_call(
        _combine_body,
        in_specs=[
            pl.BlockSpec((_NSLOT, 96, 50), lambda: (0, 0, 0)),
            pl.BlockSpec((_NAUX, 96, 50), lambda: (0, 0, 0)),
            pl.BlockSpec(memory_space=pltpu.SMEM),
        ],
        out_specs=pl.BlockSpec(memory_space=pltpu.SMEM),
        out_shape=jax.ShapeDtypeStruct((1, 1), jnp.float32),
    )(gat_t, aux_t, dense)


def kernel(output, target):
    nb = target.shape[0] * target.shape[1]
    tbt = target.reshape(nb, 50, 5).transpose(2, 0, 1)
    flat = _linearize(output.transpose(1, 0, 2))
    idx, aux = _build_idx(tbt)
    idx_pad = jnp.concatenate(
        [idx.reshape(-1), jnp.zeros((_NW * _KCH * 128 - _TOT,), jnp.int32)]
    ).reshape(_NW, _KCH, 128)
    gat = _sc_gather()(flat, idx_pad)
    gat_t = gat.reshape(-1)[:_TOT].reshape(3, _NSLOT, 96 * 50 // 150, 50).transpose(1, 0, 2, 3).reshape(_NSLOT, 96, 50)
    aux_t = aux.transpose(1, 0, 2, 3).reshape(_NAUX, 96, 50)
    dense = _dense_sum(flat)
    res = _combine(gat_t, aux_t, dense)
    return res[0, 0]
